# baseline (speedup 1.0000x reference)
import jax
import jax.numpy as jnp
from jax.experimental import pallas as pl

L = 64
NFEAT = 256
J = 256


def _rollup(x, s):
    return jnp.concatenate([x[s:], x[:s]], axis=0)


def _dot(x, w):
    return jnp.dot(x, w, preferred_element_type=jnp.float32)


def _ggcn2_kernel(x_ref, wh_ref, bh_ref, wg_ref, bg_ref, wf_ref, bf_ref,
                  out_ref):
    bh = bh_ref[...]
    bg = bg_ref[...]
    wt = wg_ref[:J, :]
    wb = wg_ref[J:, :]

    h = jnp.maximum(_dot(x_ref[...], wh_ref[...]) + bh, 0.0)

    a = _dot(h, wt)
    b = _dot(h, wb)

    abg = a + bg
    a1bg = _rollup(abg, 1)
    a2bg = _rollup(abg, 2)
    a3bg = _rollup(abg, 3)
    b1 = _rollup(b, 1)
    b2 = _rollup(b, 2)

    p1 = 0.5 * (jnp.maximum(abg + b1, 0.0) + jnp.maximum(a1bg + b, 0.0))
    p2 = 0.5 * (jnp.maximum(abg + b2, 0.0) + jnp.maximum(a2bg + b, 0.0))

    pcat = jnp.concatenate([p1, p2], axis=0)
    c = _dot(pcat, wb)
    c1 = c[:L]
    c2 = c[L:]

    fk3 = (jnp.maximum(a1bg + _rollup(c1, 2), 0.0)
           + jnp.maximum(a2bg + _rollup(c2, 1), 0.0)
           + jnp.maximum(a3bg + _rollup(c1, 1), 0.0)) * (1.0 / 3.0)

    e2 = jnp.maximum(abg + _dot(fk3, wb), 0.0)

    out_ref[...] = _dot(e2, wf_ref[...]) + bf_ref[...]


@jax.jit
def kernel(X_, W_h1, b_h1, W_g1, b_g1, W_f, b_f):
    return pl.pallas_call(
        _ggcn2_kernel,
        out_shape=jax.ShapeDtypeStruct((L, 2), jnp.float32),
    )(X_, W_h1, b_h1.reshape(1, J), W_g1, b_g1.reshape(1, J), W_f,
      b_f.reshape(1, 2))
